# baseline (speedup 1.0000x reference)
import functools

import jax
import jax.numpy as jnp
from jax import lax
from jax.experimental import pallas as pl
from jax.experimental.pallas import tpu as pltpu
from jax.experimental.pallas import tpu_sc as plsc

N = 10000
E = 320000
D = 128

NC = 2
NS = 16
NW = NC * NS
K = 128
EPW = E // NW
C = (EPW + K - 1) // K
C = C + (C % 2)
NP = 10240
RPT = NP // NS

_mesh = plsc.VectorSubcoreMesh(core_axis_name="c", subcore_axis_name="s")


NPR = NP // K


@functools.partial(
    pl.kernel,
    out_type=jax.ShapeDtypeStruct((NC, NP), jnp.float32),
    mesh=_mesh,
    scratch_types=[
        pltpu.VMEM((C, K), jnp.int32),
        pltpu.VMEM((NP,), jnp.float32),
        pltpu.VMEM((NS, RPT), jnp.float32),
        pltpu.VMEM((RPT,), jnp.float32),
        pltpu.VMEM_SHARED((NS, NP), jnp.float32),
    ],
    compiler_params=pltpu.CompilerParams(needs_layout_passes=False),
)
def _deg_kernel(dstp_hbm, zflat_hbm, out_hbm, dst_v, deg_v, blk_v, res_v, part_sp):
    c = lax.axis_index("c")
    s = lax.axis_index("s")
    wid = c * NS + s
    pltpu.sync_copy(dstp_hbm.at[wid], dst_v)
    pltpu.sync_copy(zflat_hbm, deg_v)
    ones16 = jnp.full((16,), 1.0, jnp.float32)

    def body(j, carry):
        for q in range(K // 16):
            idx = dst_v[j, pl.ds(q * 16, 16)]
            plsc.addupdate_scatter(deg_v, [idx], ones16)
        return carry

    lax.fori_loop(0, C, body, 0)
    pltpu.sync_copy(deg_v, part_sp.at[s])
    plsc.subcore_barrier()
    for r in range(NS):
        pltpu.sync_copy(part_sp.at[r, pl.ds(s * RPT, RPT)], blk_v.at[r])

    def comb(i, carry):
        tot = jnp.zeros((16,), jnp.float32)
        for r in range(NS):
            tot = tot + blk_v[r, pl.ds(i * 16, 16)]
        res_v[pl.ds(i * 16, 16)] = tot
        return carry

    lax.fori_loop(0, RPT // 16, comb, 0)
    pltpu.sync_copy(res_v, out_hbm.at[c, pl.ds(s * RPT, RPT)])


@functools.partial(
    pl.kernel,
    out_type=jax.ShapeDtypeStruct((NC, NP, D), jnp.float32),
    mesh=_mesh,
    scratch_types=[
        pltpu.VMEM((C // 2, K), jnp.int32),
        pltpu.VMEM((C // 2, K), jnp.int32),
        pltpu.VMEM((K, D), jnp.float32),
        pltpu.VMEM((K, D), jnp.float32),
        pltpu.VMEM_SHARED((NP, D), jnp.float32),
        pltpu.SemaphoreType.DMA,
        pltpu.SemaphoreType.DMA,
        pltpu.SemaphoreType.DMA,
        pltpu.SemaphoreType.DMA,
    ],
)
def _msg_kernel(g_hbm, srcp_hbm, dstp_hbm, zrows_hbm, out_hbm,
                src_v, dst_v, rows0_v, rows1_v, acc_sp,
                sem0, sem1, ssem0, ssem1):
    c = lax.axis_index("c")
    s = lax.axis_index("s")
    wid = c * NS + s
    C2 = C // 2
    pltpu.sync_copy(zrows_hbm, rows0_v)
    for i in range(RPT // K):
        pltpu.sync_copy(rows0_v, acc_sp.at[pl.ds(s * RPT + i * K, K)])
    plsc.subcore_barrier()

    for p in range(2):
        pltpu.sync_copy(srcp_hbm.at[wid, pl.ds(p * C2, C2)], src_v)
        pltpu.sync_copy(dstp_hbm.at[wid, pl.ds(p * C2, C2)], dst_v)
        pltpu.async_copy(g_hbm.at[src_v.at[0]], rows0_v, sem0)
        pltpu.async_copy(g_hbm.at[src_v.at[1]], rows1_v, sem1)

        def body(i, carry):
            j0 = 2 * i
            pltpu.make_async_copy(g_hbm.at[src_v.at[j0]], rows0_v, sem0).wait()
            sc0 = pltpu.async_copy(
                rows0_v, acc_sp.at[dst_v.at[j0]], ssem0, add=True)
            pltpu.make_async_copy(
                g_hbm.at[src_v.at[j0 + 1]], rows1_v, sem1).wait()
            sc1 = pltpu.async_copy(
                rows1_v, acc_sp.at[dst_v.at[j0 + 1]], ssem1, add=True)
            sc0.wait()

            @pl.when(i + 1 < C2 // 2)
            def _():
                pltpu.async_copy(g_hbm.at[src_v.at[j0 + 2]], rows0_v, sem0)

            sc1.wait()

            @pl.when(i + 1 < C2 // 2)
            def _():
                pltpu.async_copy(g_hbm.at[src_v.at[j0 + 3]], rows1_v, sem1)

            return carry

        lax.fori_loop(0, C2 // 2, body, 0)
    plsc.subcore_barrier()
    for i in range(RPT // K):
        r0 = s * RPT + i * K
        pltpu.sync_copy(acc_sp.at[pl.ds(r0, K)], rows0_v)
        pltpu.sync_copy(rows0_v, out_hbm.at[c, pl.ds(r0, K)])


def _dense_body(x_ref, gamma_ref, beta_ref, w_ref, degp_ref, g_ref):
    x = x_ref[...]
    mean = jnp.mean(x, axis=0, keepdims=True)
    xc = x - mean
    var = jnp.mean(xc * xc, axis=0, keepdims=True)
    h = xc * lax.rsqrt(var + 1e-5) * gamma_ref[...] + beta_ref[...]
    h = jnp.maximum(h, 0.0)
    hw = jnp.dot(h, w_ref[...], preferred_element_type=jnp.float32)
    deg = degp_ref[0, :N, :] + degp_ref[1, :N, :] + 1.0
    g_ref[...] = hw * lax.rsqrt(deg)


_dense_call = pl.pallas_call(
    _dense_body,
    out_shape=jax.ShapeDtypeStruct((N, D), jnp.float32),
)


def _combine_body(accp_ref, g_ref, degp_ref, b_ref, out_ref):
    deg = degp_ref[0, :N, :] + degp_ref[1, :N, :] + 1.0
    dinv = lax.rsqrt(deg)
    acc = accp_ref[0, :N, :] + accp_ref[1, :N, :] + g_ref[...]
    out_ref[...] = acc * dinv + b_ref[...]


_combine_call = pl.pallas_call(
    _combine_body,
    out_shape=jax.ShapeDtypeStruct((N, D), jnp.float32),
)


@jax.jit
def kernel(x, edge_index, W, b, gamma, beta):
    pad = C * K - EPW
    src = edge_index[0].reshape(NW, EPW)
    dst = edge_index[1].reshape(NW, EPW)
    srcp = jnp.pad(src, ((0, 0), (0, pad)), constant_values=0)
    dstp = jnp.pad(dst, ((0, 0), (0, pad)), constant_values=NP - 1)
    srcp = srcp.reshape(NW, C, K)
    dstp = dstp.reshape(NW, C, K)
    zeros_rows = jnp.zeros((K, D), jnp.float32)
    zeros_flat = jnp.zeros((NP,), jnp.float32)

    degp = _deg_kernel(dstp, zeros_flat).reshape(NC, NP, 1)
    g = _dense_call(x, gamma.reshape(1, D), beta.reshape(1, D), W, degp)
    accp = _msg_kernel(g, srcp, dstp, zeros_rows)
    return _combine_call(accp, g, degp, b.reshape(1, D))


# speedup vs baseline: 1.6683x; 1.6683x over previous
import functools

import jax
import jax.numpy as jnp
from jax import lax
from jax.experimental import pallas as pl
from jax.experimental.pallas import tpu as pltpu
from jax.experimental.pallas import tpu_sc as plsc

N = 10000
E = 320000
D = 128

NC = 2
NS = 16
NW = NC * NS
K = 128
EPW = E // NW
C = 80
NP = 10240
RPT = NP // NS
H = NP // 2
HT = H + 8
RT = H // NS
CAPC = 24
CAP = CAPC * K
GROUPS = ((0, 0), (1, 0), (0, 1), (1, 1))

_mesh = plsc.VectorSubcoreMesh(core_axis_name="c", subcore_axis_name="s")


@functools.partial(
    pl.kernel,
    out_type=(
        jax.ShapeDtypeStruct((NC, NP), jnp.float32),
        jax.ShapeDtypeStruct((NW * 8, CAP), jnp.int32),
    ),
    mesh=_mesh,
    scratch_types=[
        pltpu.VMEM((C, K), jnp.int32),
        pltpu.VMEM((C, K), jnp.int32),
        pltpu.VMEM((NP,), jnp.float32),
        pltpu.VMEM((NS, RPT), jnp.float32),
        pltpu.VMEM((RPT,), jnp.float32),
    ] + [pltpu.VMEM((CAP,), jnp.int32)] * 8
    + [pltpu.VMEM_SHARED((NS, NP), jnp.float32)],
    compiler_params=pltpu.CompilerParams(needs_layout_passes=False),
)
def _deg_kernel(srcp_hbm, dstp_hbm, zflat_hbm, deg_hbm, lists_hbm,
                src_v, dst_v, deg_v, blk_v, res_v,
                sl0, dl0, sl1, dl1, sl2, dl2, sl3, dl3, part_sp):
    c = lax.axis_index("c")
    s = lax.axis_index("s")
    wid = c * NS + s
    src_lists = (sl0, sl1, sl2, sl3)
    dst_lists = (dl0, dl1, dl2, dl3)
    pltpu.sync_copy(srcp_hbm.at[wid], src_v)
    pltpu.sync_copy(dstp_hbm.at[wid], dst_v)
    pltpu.sync_copy(zflat_hbm, deg_v)

    zero16 = jnp.zeros((16,), jnp.int32)
    pad16 = jnp.full((16,), H, jnp.int32)

    def fill(i, carry):
        for gi in range(4):
            src_lists[gi][pl.ds(i * 16, 16)] = zero16
            dst_lists[gi][pl.ds(i * 16, 16)] = pad16
        return carry

    lax.fori_loop(0, CAP // 16, fill, 0)

    ones16 = jnp.full((16,), 1.0, jnp.float32)

    def body(j, cnt):
        cnt = list(cnt)
        for q in range(K // 16):
            dstv = dst_v[j, pl.ds(q * 16, 16)]
            srcv = src_v[j, pl.ds(q * 16, 16)]
            plsc.addupdate_scatter(deg_v, [dstv], ones16)
            real = dstv < N
            src_hi = srcv >= H
            dst_hi = dstv >= H
            for gi, (aa, bb) in enumerate(GROUPS):
                m = real
                m = jnp.logical_and(m, src_hi if aa else jnp.logical_not(src_hi))
                m = jnp.logical_and(m, dst_hi if bb else jnp.logical_not(dst_hi))
                pm = m.astype(jnp.int32)
                pos = plsc.cumsum(pm) - 1 + cnt[gi]
                plsc.store_scatter(src_lists[gi], [pos], srcv - aa * H, mask=m)
                plsc.store_scatter(dst_lists[gi], [pos], dstv - bb * H, mask=m)
                cnt[gi] = cnt[gi] + jnp.sum(pm)
        return tuple(cnt)

    z = jnp.int32(0)
    lax.fori_loop(0, C, body, (z, z, z, z))
    for gi in range(4):
        pltpu.sync_copy(src_lists[gi], lists_hbm.at[wid * 8 + 2 * gi])
        pltpu.sync_copy(dst_lists[gi], lists_hbm.at[wid * 8 + 2 * gi + 1])

    pltpu.sync_copy(deg_v, part_sp.at[s])
    plsc.subcore_barrier()
    for r in range(NS):
        pltpu.sync_copy(part_sp.at[r, pl.ds(s * RPT, RPT)], blk_v.at[r])

    def comb(i, carry):
        tot = jnp.zeros((16,), jnp.float32)
        for r in range(NS):
            tot = tot + blk_v[r, pl.ds(i * 16, 16)]
        res_v[pl.ds(i * 16, 16)] = tot
        return carry

    lax.fori_loop(0, RPT // 16, comb, 0)
    pltpu.sync_copy(res_v, deg_hbm.at[c, pl.ds(s * RPT, RPT)])


_STAGE = ((0, 128), (128, 128), (256, 64))


@functools.partial(
    pl.kernel,
    out_type=jax.ShapeDtypeStruct((NC * 2, H, D), jnp.float32),
    mesh=_mesh,
    scratch_types=[
        pltpu.VMEM((CAPC, K), jnp.int32),
        pltpu.VMEM((CAPC, K), jnp.int32),
        pltpu.VMEM((K, D), jnp.float32),
        pltpu.VMEM((K, D), jnp.float32),
        pltpu.VMEM_SHARED((H, D), jnp.float32),
        pltpu.VMEM_SHARED((HT, D), jnp.float32),
        pltpu.SemaphoreType.DMA,
        pltpu.SemaphoreType.DMA,
    ],
)
def _msg_kernel(gpad_hbm, lists_hbm, zrows_hbm, out_hbm,
                srcl_v, dstl_v, rows0_v, rows1_v, g_sp, acc_sp, sem0, sem1):
    c = lax.axis_index("c")
    s = lax.axis_index("s")
    wid = c * NS + s
    for b in (0, 1):
        pltpu.sync_copy(zrows_hbm, rows0_v)
        for off, sz in _STAGE:
            pltpu.sync_copy(rows0_v.at[pl.ds(0, sz)],
                            acc_sp.at[pl.ds(s * RT + off, sz)])

        @pl.when(s == NS - 1)
        def _():
            pltpu.sync_copy(rows0_v.at[pl.ds(0, HT - H)],
                            acc_sp.at[pl.ds(H, HT - H)])

        plsc.subcore_barrier()
        for a in (0, 1):
            for off, sz in _STAGE:
                pltpu.sync_copy(gpad_hbm.at[pl.ds(a * H + s * RT + off, sz)],
                                rows0_v.at[pl.ds(0, sz)])
                pltpu.sync_copy(rows0_v.at[pl.ds(0, sz)],
                                g_sp.at[pl.ds(s * RT + off, sz)])
            plsc.subcore_barrier()
            gi = 2 * b + a
            pltpu.sync_copy(lists_hbm.at[wid * 8 + 2 * gi], srcl_v)
            pltpu.sync_copy(lists_hbm.at[wid * 8 + 2 * gi + 1], dstl_v)

            pltpu.async_copy(g_sp.at[srcl_v.at[0]], rows0_v, sem0)
            pltpu.async_copy(g_sp.at[srcl_v.at[1]], rows1_v, sem1)

            def body(i, carry):
                j0 = 2 * i
                pltpu.make_async_copy(
                    g_sp.at[srcl_v.at[j0]], rows0_v, sem0).wait()
                pltpu.sync_copy(rows0_v, acc_sp.at[dstl_v.at[j0]], add=True)

                @pl.when(i + 1 < CAPC // 2)
                def _():
                    pltpu.async_copy(
                        g_sp.at[srcl_v.at[j0 + 2]], rows0_v, sem0)

                pltpu.make_async_copy(
                    g_sp.at[srcl_v.at[j0 + 1]], rows1_v, sem1).wait()
                pltpu.sync_copy(
                    rows1_v, acc_sp.at[dstl_v.at[j0 + 1]], add=True)

                @pl.when(i + 1 < CAPC // 2)
                def _():
                    pltpu.async_copy(
                        g_sp.at[srcl_v.at[j0 + 3]], rows1_v, sem1)

                return carry

            lax.fori_loop(0, CAPC // 2, body, 0)
            plsc.subcore_barrier()
        for off, sz in _STAGE:
            pltpu.sync_copy(acc_sp.at[pl.ds(s * RT + off, sz)],
                            rows0_v.at[pl.ds(0, sz)])
            pltpu.sync_copy(rows0_v.at[pl.ds(0, sz)],
                            out_hbm.at[c * 2 + b, pl.ds(s * RT + off, sz)])
        plsc.subcore_barrier()


def _dense_body(x_ref, gamma_ref, beta_ref, w_ref, degp_ref, g_ref):
    x = x_ref[...]
    mean = jnp.mean(x, axis=0, keepdims=True)
    xc = x - mean
    var = jnp.mean(xc * xc, axis=0, keepdims=True)
    h = xc * lax.rsqrt(var + 1e-5) * gamma_ref[...] + beta_ref[...]
    h = jnp.maximum(h, 0.0)
    hw = jnp.dot(h, w_ref[...], preferred_element_type=jnp.float32)
    deg = degp_ref[0, :N, :] + degp_ref[1, :N, :] + 1.0
    g_ref[pl.ds(0, N), :] = hw * lax.rsqrt(deg)
    g_ref[pl.ds(N, NP - N), :] = jnp.zeros((NP - N, D), jnp.float32)


_dense_call = pl.pallas_call(
    _dense_body,
    out_shape=jax.ShapeDtypeStruct((NP, D), jnp.float32),
)


def _combine_body(accp_ref, g_ref, degp_ref, b_ref, out_ref):
    deg = degp_ref[0, :N, :] + degp_ref[1, :N, :] + 1.0
    dinv = lax.rsqrt(deg)
    bb = b_ref[...]
    acc_t = accp_ref[0] + accp_ref[2] + g_ref[pl.ds(0, H), :]
    out_ref[pl.ds(0, H), :] = acc_t * dinv[:H, :] + bb
    RB = N - H
    acc_b = (accp_ref[1, pl.ds(0, RB), :] + accp_ref[3, pl.ds(0, RB), :]
             + g_ref[pl.ds(H, RB), :])
    out_ref[pl.ds(H, RB), :] = acc_b * dinv[H:, :] + bb


_combine_call = pl.pallas_call(
    _combine_body,
    out_shape=jax.ShapeDtypeStruct((N, D), jnp.float32),
)


@jax.jit
def kernel(x, edge_index, W, b, gamma, beta):
    pad = C * K - EPW
    src = edge_index[0].reshape(NW, EPW)
    dst = edge_index[1].reshape(NW, EPW)
    srcp = jnp.pad(src, ((0, 0), (0, pad)), constant_values=0)
    dstp = jnp.pad(dst, ((0, 0), (0, pad)), constant_values=NP - 1)
    srcp = srcp.reshape(NW, C, K)
    dstp = dstp.reshape(NW, C, K)
    zeros_rows = jnp.zeros((K, D), jnp.float32)
    zeros_flat = jnp.zeros((NP,), jnp.float32)

    degp, lists = _deg_kernel(srcp, dstp, zeros_flat)
    degp = degp.reshape(NC, NP, 1)
    gpad = _dense_call(x, gamma.reshape(1, D), beta.reshape(1, D), W, degp)
    accp = _msg_kernel(gpad, lists.reshape(NW * 8, CAPC, K), zeros_rows)
    return _combine_call(accp, gpad, degp, b.reshape(1, D))


# speedup vs baseline: 1.7328x; 1.0386x over previous
import functools

import jax
import jax.numpy as jnp
from jax import lax
from jax.experimental import pallas as pl
from jax.experimental.pallas import tpu as pltpu
from jax.experimental.pallas import tpu_sc as plsc

N = 10000
E = 320000
D = 128

NC = 2
NS = 16
NW = NC * NS
K = 128
EPW = E // NW
C = 80
NP = 10240
RPT = NP // NS
H = NP // 2
HT = H + 8
RT = H // NS
CAPC = 24
CAP = CAPC * K
GROUPS = ((0, 0), (1, 0), (0, 1), (1, 1))

_mesh = plsc.VectorSubcoreMesh(core_axis_name="c", subcore_axis_name="s")


@functools.partial(
    pl.kernel,
    out_type=(
        jax.ShapeDtypeStruct((NC, NP), jnp.float32),
        jax.ShapeDtypeStruct((NW * 8, CAP), jnp.int32),
    ),
    mesh=_mesh,
    scratch_types=[
        pltpu.VMEM((C, K), jnp.int32),
        pltpu.VMEM((C, K), jnp.int32),
        pltpu.VMEM((NP,), jnp.float32),
        pltpu.VMEM((NS, RPT), jnp.float32),
        pltpu.VMEM((RPT,), jnp.float32),
    ] + [pltpu.VMEM((CAP,), jnp.int32)] * 8
    + [pltpu.VMEM_SHARED((NS, NP), jnp.float32)],
    compiler_params=pltpu.CompilerParams(needs_layout_passes=False),
)
def _deg_kernel(srcp_hbm, dstp_hbm, zflat_hbm, deg_hbm, lists_hbm,
                src_v, dst_v, deg_v, blk_v, res_v,
                sl0, dl0, sl1, dl1, sl2, dl2, sl3, dl3, part_sp):
    c = lax.axis_index("c")
    s = lax.axis_index("s")
    wid = c * NS + s
    src_lists = (sl0, sl1, sl2, sl3)
    dst_lists = (dl0, dl1, dl2, dl3)
    pltpu.sync_copy(srcp_hbm.at[wid], src_v)
    pltpu.sync_copy(dstp_hbm.at[wid], dst_v)
    pltpu.sync_copy(zflat_hbm, deg_v)

    zero16 = jnp.zeros((16,), jnp.int32)
    pad16 = jnp.full((16,), H, jnp.int32)

    def fill(i, carry):
        for gi in range(4):
            src_lists[gi][pl.ds(i * 16, 16)] = zero16
            dst_lists[gi][pl.ds(i * 16, 16)] = pad16
        return carry

    lax.fori_loop(0, CAP // 16, fill, 0)

    ones16 = jnp.full((16,), 1.0, jnp.float32)

    def body(j, cnt):
        cnt = list(cnt)
        for q in range(K // 16):
            dstv = dst_v[j, pl.ds(q * 16, 16)]
            srcv = src_v[j, pl.ds(q * 16, 16)]
            plsc.addupdate_scatter(deg_v, [dstv], ones16)
            real = dstv < N
            src_hi = srcv >= H
            dst_hi = dstv >= H
            for gi, (aa, bb) in enumerate(GROUPS):
                m = real
                m = jnp.logical_and(m, src_hi if aa else jnp.logical_not(src_hi))
                m = jnp.logical_and(m, dst_hi if bb else jnp.logical_not(dst_hi))
                pm = m.astype(jnp.int32)
                pos = plsc.cumsum(pm) - 1 + cnt[gi]
                plsc.store_scatter(src_lists[gi], [pos], srcv - aa * H, mask=m)
                plsc.store_scatter(dst_lists[gi], [pos], dstv - bb * H, mask=m)
                cnt[gi] = cnt[gi] + jnp.sum(pm)
        return tuple(cnt)

    z = jnp.int32(0)
    lax.fori_loop(0, C, body, (z, z, z, z))
    for gi in range(4):
        pltpu.sync_copy(src_lists[gi], lists_hbm.at[wid * 8 + 2 * gi])
        pltpu.sync_copy(dst_lists[gi], lists_hbm.at[wid * 8 + 2 * gi + 1])

    pltpu.sync_copy(deg_v, part_sp.at[s])
    plsc.subcore_barrier()
    for r in range(NS):
        pltpu.sync_copy(part_sp.at[r, pl.ds(s * RPT, RPT)], blk_v.at[r])

    def comb(i, carry):
        tot = jnp.zeros((16,), jnp.float32)
        for r in range(NS):
            tot = tot + blk_v[r, pl.ds(i * 16, 16)]
        res_v[pl.ds(i * 16, 16)] = tot
        return carry

    lax.fori_loop(0, RPT // 16, comb, 0)
    pltpu.sync_copy(res_v, deg_hbm.at[c, pl.ds(s * RPT, RPT)])


_STAGE = ((0, 128), (128, 128), (256, 64))


@functools.partial(
    pl.kernel,
    out_type=jax.ShapeDtypeStruct((NC * 2, H, D), jnp.float32),
    mesh=_mesh,
    scratch_types=[
        pltpu.VMEM((CAPC, K), jnp.int32),
        pltpu.VMEM((CAPC, K), jnp.int32),
        pltpu.VMEM((K, D), jnp.float32),
        pltpu.VMEM((K, D), jnp.float32),
        pltpu.VMEM_SHARED((H, D), jnp.float32),
        pltpu.VMEM_SHARED((HT, D), jnp.float32),
        pltpu.SemaphoreType.DMA,
        pltpu.SemaphoreType.DMA,
        pltpu.SemaphoreType.DMA,
        pltpu.SemaphoreType.DMA,
    ],
)
def _msg_kernel(gpad_hbm, lists_hbm, zrows_hbm, out_hbm,
                srcl_v, dstl_v, rows0_v, rows1_v, g_sp, acc_sp,
                sem0, sem1, sem2, sem3):
    c = lax.axis_index("c")
    s = lax.axis_index("s")
    wid = c * NS + s
    for b in (0, 1):
        pltpu.sync_copy(zrows_hbm, rows0_v)
        zd = []
        for off, sz in _STAGE:
            zd.append(pltpu.async_copy(
                rows0_v.at[pl.ds(0, sz)],
                acc_sp.at[pl.ds(s * RT + off, sz)], sem2))

        @pl.when(s == NS - 1)
        def _():
            pltpu.async_copy(rows0_v.at[pl.ds(0, HT - H)],
                             acc_sp.at[pl.ds(H, HT - H)], sem3).wait()

        for d in zd:
            d.wait()
        plsc.subcore_barrier()
        for a in (0, 1):
            (o0, z0), (o1, z1), (o2, z2) = _STAGE
            base = a * H + s * RT
            l0 = pltpu.async_copy(gpad_hbm.at[pl.ds(base + o0, z0)],
                                  rows0_v.at[pl.ds(0, z0)], sem0)
            l1 = pltpu.async_copy(gpad_hbm.at[pl.ds(base + o1, z1)],
                                  rows1_v.at[pl.ds(0, z1)], sem1)
            l0.wait()
            s0 = pltpu.async_copy(rows0_v.at[pl.ds(0, z0)],
                                  g_sp.at[pl.ds(s * RT + o0, z0)], sem2)
            l1.wait()
            s1 = pltpu.async_copy(rows1_v.at[pl.ds(0, z1)],
                                  g_sp.at[pl.ds(s * RT + o1, z1)], sem3)
            s0.wait()
            l2 = pltpu.async_copy(gpad_hbm.at[pl.ds(base + o2, z2)],
                                  rows0_v.at[pl.ds(0, z2)], sem0)
            l2.wait()
            s2 = pltpu.async_copy(rows0_v.at[pl.ds(0, z2)],
                                  g_sp.at[pl.ds(s * RT + o2, z2)], sem2)
            s1.wait()
            s2.wait()
            plsc.subcore_barrier()
            gi = 2 * b + a
            pltpu.sync_copy(lists_hbm.at[wid * 8 + 2 * gi], srcl_v)
            pltpu.sync_copy(lists_hbm.at[wid * 8 + 2 * gi + 1], dstl_v)

            pltpu.async_copy(g_sp.at[srcl_v.at[0]], rows0_v, sem0)
            pltpu.async_copy(g_sp.at[srcl_v.at[1]], rows1_v, sem1)

            def body(i, carry):
                j0 = 2 * i
                pltpu.make_async_copy(
                    g_sp.at[srcl_v.at[j0]], rows0_v, sem0).wait()
                pltpu.sync_copy(rows0_v, acc_sp.at[dstl_v.at[j0]], add=True)

                @pl.when(i + 1 < CAPC // 2)
                def _():
                    pltpu.async_copy(
                        g_sp.at[srcl_v.at[j0 + 2]], rows0_v, sem0)

                pltpu.make_async_copy(
                    g_sp.at[srcl_v.at[j0 + 1]], rows1_v, sem1).wait()
                pltpu.sync_copy(
                    rows1_v, acc_sp.at[dstl_v.at[j0 + 1]], add=True)

                @pl.when(i + 1 < CAPC // 2)
                def _():
                    pltpu.async_copy(
                        g_sp.at[srcl_v.at[j0 + 3]], rows1_v, sem1)

                return carry

            lax.fori_loop(0, CAPC // 2, body, 0)
            plsc.subcore_barrier()
        (o0, z0), (o1, z1), (o2, z2) = _STAGE
        w0 = pltpu.async_copy(acc_sp.at[pl.ds(s * RT + o0, z0)],
                              rows0_v.at[pl.ds(0, z0)], sem0)
        w1 = pltpu.async_copy(acc_sp.at[pl.ds(s * RT + o1, z1)],
                              rows1_v.at[pl.ds(0, z1)], sem1)
        w0.wait()
        p0 = pltpu.async_copy(rows0_v.at[pl.ds(0, z0)],
                              out_hbm.at[c * 2 + b, pl.ds(s * RT + o0, z0)],
                              sem2)
        w1.wait()
        p1 = pltpu.async_copy(rows1_v.at[pl.ds(0, z1)],
                              out_hbm.at[c * 2 + b, pl.ds(s * RT + o1, z1)],
                              sem3)
        p0.wait()
        w2 = pltpu.async_copy(acc_sp.at[pl.ds(s * RT + o2, z2)],
                              rows0_v.at[pl.ds(0, z2)], sem0)
        w2.wait()
        p2 = pltpu.async_copy(rows0_v.at[pl.ds(0, z2)],
                              out_hbm.at[c * 2 + b, pl.ds(s * RT + o2, z2)],
                              sem2)
        p1.wait()
        p2.wait()
        plsc.subcore_barrier()


def _dense_body(x_ref, gamma_ref, beta_ref, w_ref, degp_ref, g_ref):
    x = x_ref[...]
    mean = jnp.mean(x, axis=0, keepdims=True)
    xc = x - mean
    var = jnp.mean(xc * xc, axis=0, keepdims=True)
    h = xc * lax.rsqrt(var + 1e-5) * gamma_ref[...] + beta_ref[...]
    h = jnp.maximum(h, 0.0)
    hw = jnp.dot(h, w_ref[...], preferred_element_type=jnp.float32)
    deg = degp_ref[0, :N, :] + degp_ref[1, :N, :] + 1.0
    g_ref[pl.ds(0, N), :] = hw * lax.rsqrt(deg)
    g_ref[pl.ds(N, NP - N), :] = jnp.zeros((NP - N, D), jnp.float32)


_dense_call = pl.pallas_call(
    _dense_body,
    out_shape=jax.ShapeDtypeStruct((NP, D), jnp.float32),
)


def _combine_body(accp_ref, g_ref, degp_ref, b_ref, out_ref):
    deg = degp_ref[0, :N, :] + degp_ref[1, :N, :] + 1.0
    dinv = lax.rsqrt(deg)
    bb = b_ref[...]
    acc_t = accp_ref[0] + accp_ref[2] + g_ref[pl.ds(0, H), :]
    out_ref[pl.ds(0, H), :] = acc_t * dinv[:H, :] + bb
    RB = N - H
    acc_b = (accp_ref[1, pl.ds(0, RB), :] + accp_ref[3, pl.ds(0, RB), :]
             + g_ref[pl.ds(H, RB), :])
    out_ref[pl.ds(H, RB), :] = acc_b * dinv[H:, :] + bb


_combine_call = pl.pallas_call(
    _combine_body,
    out_shape=jax.ShapeDtypeStruct((N, D), jnp.float32),
)


@jax.jit
def kernel(x, edge_index, W, b, gamma, beta):
    pad = C * K - EPW
    src = edge_index[0].reshape(NW, EPW)
    dst = edge_index[1].reshape(NW, EPW)
    srcp = jnp.pad(src, ((0, 0), (0, pad)), constant_values=0)
    dstp = jnp.pad(dst, ((0, 0), (0, pad)), constant_values=NP - 1)
    srcp = srcp.reshape(NW, C, K)
    dstp = dstp.reshape(NW, C, K)
    zeros_rows = jnp.zeros((K, D), jnp.float32)
    zeros_flat = jnp.zeros((NP,), jnp.float32)

    degp, lists = _deg_kernel(srcp, dstp, zeros_flat)
    degp = degp.reshape(NC, NP, 1)
    gpad = _dense_call(x, gamma.reshape(1, D), beta.reshape(1, D), W, degp)
    accp = _msg_kernel(gpad, lists.reshape(NW * 8, CAPC, K), zeros_rows)
    return _combine_call(accp, gpad, degp, b.reshape(1, D))
